# baseline (speedup 1.0000x reference)
import functools

import jax
import jax.numpy as jnp
import numpy as np
from jax.experimental import pallas as pl
from jax.experimental.pallas import tpu as pltpu
from jax.experimental.pallas import tpu_sc as plsc

_COMBS = ((0, 3), (1, 3), (2, 3))
_WINDOW = 256
_BLK = 2048
_BLKP = 16384
_NCHUNK = 8


def _pack_plane(plane):
    t = jnp.transpose(plane, (1, 2, 0))
    tx = jnp.concatenate([t[:, 1:], t[:, -1:]], axis=1)
    ty = jnp.concatenate([t[1:], t[-1:]], axis=0)
    txy = jnp.concatenate([tx[1:], tx[-1:]], axis=0)
    H, W, C = t.shape
    return jnp.concatenate([t, tx, ty, txy], axis=-1).reshape(H * W, 4 * C)


def _prep(cT, plane_shapes):
    P = cT.shape[1]

    def body(c_ref, w_ref, i_ref):
        w_rows = []
        i_rows = []
        for ci, comb in enumerate(_COMBS):
            _, H, W = plane_shapes[ci]
            cx = c_ref[comb[0]:comb[0] + 1, :]
            cy = c_ref[comb[1]:comb[1] + 1, :]
            x = jnp.clip((cx + 1.0) * (0.5 * (W - 1)), 0.0, W - 1)
            y = jnp.clip((cy + 1.0) * (0.5 * (H - 1)), 0.0, H - 1)
            x0 = jnp.floor(x)
            y0 = jnp.floor(y)
            i_rows.append(y0.astype(jnp.int32) * W + x0.astype(jnp.int32))
            wx = x - x0
            wy = y - y0
            w_rows += [(1.0 - wx) * (1.0 - wy), wx * (1.0 - wy),
                       (1.0 - wx) * wy, wx * wy]
        w_rows.append(jnp.zeros((4, w_rows[0].shape[1]), jnp.float32))
        w_ref[...] = jnp.concatenate(w_rows, axis=0).astype(jnp.bfloat16)
        i_ref[...] = jnp.concatenate(i_rows, axis=0)

    return pl.pallas_call(
        body,
        grid=(P // _BLKP,),
        in_specs=[pl.BlockSpec((cT.shape[0], _BLKP), lambda i: (0, i))],
        out_specs=[
            pl.BlockSpec((16, _BLKP), lambda i: (0, i)),
            pl.BlockSpec((3, _BLKP), lambda i: (0, i)),
        ],
        out_shape=[
            jax.ShapeDtypeStruct((16, P), jnp.bfloat16),
            jax.ShapeDtypeStruct((3, P), jnp.int32),
        ],
    )(cT)


def _sc_gather3(tables, idx3, pbase, pcount):
    d = tables[0].shape[1]
    mesh = plsc.VectorSubcoreMesh(core_axis_name="c", subcore_axis_name="s")
    ibase = pbase // _WINDOW

    @functools.partial(
        pl.kernel,
        out_type=jax.ShapeDtypeStruct((3 * pcount, d), tables[0].dtype),
        mesh=mesh,
    )
    def gather_kernel(t0_hbm, t1_hbm, t2_hbm, idx_hbm, out_hbm):
        for ci, t_hbm in enumerate((t0_hbm, t1_hbm, t2_hbm)):
            def body(i_vmem, o_vmem, t_hbm=t_hbm):
                pltpu.sync_copy(t_hbm.at[i_vmem.at[0]], o_vmem)

            obase = ci * (pcount // _WINDOW)
            pltpu.emit_pipeline(
                body,
                grid=(pcount // _WINDOW,),
                in_specs=[pl.BlockSpec((1, _WINDOW),
                                       lambda i, ci=ci: (ci, ibase + i))],
                out_specs=[pl.BlockSpec((_WINDOW, d),
                                        lambda i, obase=obase: (obase + i, 0))],
                core_axis_name=("c", "s"),
                dimension_semantics=(pltpu.PARALLEL,),
            )(idx_hbm, out_hbm)

    return gather_kernel(*tables, idx3)


def _combine(w16, g3, E, S, pbase, prev):
    P = w16.shape[1]
    pcount = g3.shape[1]
    D = g3.shape[2]
    C = S.shape[1]
    wbase = pbase // _BLK

    def body(prev_ref, w_ref, g_ref, E_ref, S_ref, o_ref):
        del prev_ref
        wt = jnp.transpose(w_ref[...], (1, 0))
        wpat = jax.lax.dot_general(
            wt, E_ref[...], (((1,), (0,)), ((), ())),
            preferred_element_type=jnp.float32)
        acc = g_ref[0] * wpat[:, 0:D]
        acc = acc + g_ref[1] * wpat[:, D:2 * D]
        acc = acc + g_ref[2] * wpat[:, 2 * D:3 * D]
        o_ref[...] = jax.lax.dot_general(
            acc.astype(jnp.bfloat16), S_ref[...], (((1,), (0,)), ((), ())),
            preferred_element_type=jnp.float32)

    return pl.pallas_call(
        body,
        grid=(pcount // _BLK,),
        in_specs=[
            pl.BlockSpec(memory_space=pl.ANY),
            pl.BlockSpec((16, _BLK), lambda i: (0, wbase + i)),
            pl.BlockSpec((3, _BLK, D), lambda i: (0, i, 0)),
            pl.BlockSpec(E.shape, lambda i: (0, 0)),
            pl.BlockSpec(S.shape, lambda i: (0, 0)),
        ],
        out_specs=pl.BlockSpec((_BLK, C), lambda i: (wbase + i, 0)),
        out_shape=jax.ShapeDtypeStruct((P, C), jnp.float32),
        input_output_aliases={0: 0},
    )(prev, w16, g3, E, S)


def kernel(inp, plane0, plane1, plane2):
    planes = (plane0, plane1, plane2)
    P = inp.shape[0]
    C = plane0.shape[0]
    D = 4 * C
    tables = [_pack_plane(p) for p in planes]
    cT = inp.T
    w16, idx3 = _prep(cT, [p.shape for p in planes])

    E = np.zeros((16, 3 * D), np.float32)
    for ci in range(3):
        for c in range(4):
            E[4 * ci + c, ci * D + c * C:ci * D + (c + 1) * C] = 1.0
    S = np.zeros((D, C), np.float32)
    for c in range(4):
        S[c * C:(c + 1) * C, :] += np.eye(C, dtype=np.float32)
    E = jnp.asarray(E, jnp.bfloat16)
    S = jnp.asarray(S, jnp.bfloat16)

    pc = P // _NCHUNK
    out = jnp.zeros((P, C), jnp.float32)
    for k in range(_NCHUNK):
        g = _sc_gather3(tables, idx3, k * pc, pc)
        out = _combine(w16, g.reshape(3, pc, D), E, S, k * pc, out)
    return out


# speedup vs baseline: 1.0766x; 1.0766x over previous
import functools

import jax
import jax.numpy as jnp
import numpy as np
from jax.experimental import pallas as pl
from jax.experimental.pallas import tpu as pltpu
from jax.experimental.pallas import tpu_sc as plsc

_COMBS = ((0, 3), (1, 3), (2, 3))
_WINDOW = 256
_BLK = 2048
_BLKP = 16384
_NCHUNK = 8


def _pack_plane(plane):
    t = jnp.transpose(plane, (1, 2, 0))
    tx = jnp.concatenate([t[:, 1:], t[:, -1:]], axis=1)
    ty = jnp.concatenate([t[1:], t[-1:]], axis=0)
    txy = jnp.concatenate([tx[1:], tx[-1:]], axis=0)
    H, W, C = t.shape
    return jnp.concatenate([t, tx, ty, txy], axis=-1).reshape(H * W, 4 * C)


def _prep(cT, plane_shapes):
    P = cT.shape[1]

    def body(c_ref, w_ref, i_ref):
        w_rows = []
        i_rows = []
        for ci, comb in enumerate(_COMBS):
            _, H, W = plane_shapes[ci]
            cx = c_ref[comb[0]:comb[0] + 1, :]
            cy = c_ref[comb[1]:comb[1] + 1, :]
            x = jnp.clip((cx + 1.0) * (0.5 * (W - 1)), 0.0, W - 1)
            y = jnp.clip((cy + 1.0) * (0.5 * (H - 1)), 0.0, H - 1)
            x0 = jnp.floor(x)
            y0 = jnp.floor(y)
            i_rows.append(y0.astype(jnp.int32) * W + x0.astype(jnp.int32))
            wx = x - x0
            wy = y - y0
            w_rows += [(1.0 - wx) * (1.0 - wy), wx * (1.0 - wy),
                       (1.0 - wx) * wy, wx * wy]
        w_rows.append(jnp.zeros((4, w_rows[0].shape[1]), jnp.float32))
        w_ref[...] = jnp.concatenate(w_rows, axis=0).astype(jnp.bfloat16)
        i_ref[...] = jnp.concatenate(i_rows, axis=0)

    return pl.pallas_call(
        body,
        grid=(P // _BLKP,),
        in_specs=[pl.BlockSpec((cT.shape[0], _BLKP), lambda i: (0, i))],
        out_specs=[
            pl.BlockSpec((16, _BLKP), lambda i: (0, i)),
            pl.BlockSpec((3, _BLKP), lambda i: (0, i)),
        ],
        out_shape=[
            jax.ShapeDtypeStruct((16, P), jnp.bfloat16),
            jax.ShapeDtypeStruct((3, P), jnp.int32),
        ],
    )(cT)


def _sc_gather3(tables, idx3, pbase, pcount):
    d = tables[0].shape[1]
    mesh = plsc.VectorSubcoreMesh(core_axis_name="c", subcore_axis_name="s")
    ibase = pbase // _WINDOW

    @functools.partial(
        pl.kernel,
        out_type=jax.ShapeDtypeStruct((3 * pcount, d), tables[0].dtype),
        mesh=mesh,
    )
    def gather_kernel(t0_hbm, t1_hbm, t2_hbm, idx_hbm, out_hbm):
        for ci, t_hbm in enumerate((t0_hbm, t1_hbm, t2_hbm)):
            def body(i_vmem, o_vmem, t_hbm=t_hbm):
                pltpu.sync_copy(t_hbm.at[i_vmem.at[0]], o_vmem)

            obase = ci * (pcount // _WINDOW)
            pltpu.emit_pipeline(
                body,
                grid=(pcount // _WINDOW,),
                in_specs=[pl.BlockSpec((1, _WINDOW),
                                       lambda i, ci=ci: (ci, ibase + i))],
                out_specs=[pl.BlockSpec((_WINDOW, d),
                                        lambda i, obase=obase: (obase + i, 0))],
                core_axis_name=("c", "s"),
                dimension_semantics=(pltpu.PARALLEL,),
            )(idx_hbm, out_hbm)

    return gather_kernel(*tables, idx3)


def _combine(w16, g3, E, S, pbase, prev):
    P = w16.shape[1]
    pcount = g3.shape[1]
    D = g3.shape[2]
    C = S.shape[1]
    wbase = pbase // _BLK

    def body(*refs):
        (w_ref, g_ref, E_ref, S_ref, o_ref) = refs[-5:]
        wt = jnp.transpose(w_ref[...], (1, 0))
        wpat = jax.lax.dot_general(
            wt, E_ref[...], (((1,), (0,)), ((), ())),
            preferred_element_type=jnp.float32)
        acc = g_ref[0] * wpat[:, 0:D]
        acc = acc + g_ref[1] * wpat[:, D:2 * D]
        acc = acc + g_ref[2] * wpat[:, 2 * D:3 * D]
        o_ref[...] = jax.lax.dot_general(
            acc.astype(jnp.bfloat16), S_ref[...], (((1,), (0,)), ((), ())),
            preferred_element_type=jnp.float32)

    in_specs = [
        pl.BlockSpec((16, _BLK), lambda i: (0, wbase + i)),
        pl.BlockSpec((3, _BLK, D), lambda i: (0, i, 0)),
        pl.BlockSpec(E.shape, lambda i: (0, 0)),
        pl.BlockSpec(S.shape, lambda i: (0, 0)),
    ]
    args = (w16, g3, E, S)
    aliases = {}
    if prev is not None:
        in_specs = [pl.BlockSpec(memory_space=pl.ANY)] + in_specs
        args = (prev,) + args
        aliases = {0: 0}
    return pl.pallas_call(
        body,
        grid=(pcount // _BLK,),
        in_specs=in_specs,
        out_specs=pl.BlockSpec((_BLK, C), lambda i: (wbase + i, 0)),
        out_shape=jax.ShapeDtypeStruct((P, C), jnp.float32),
        input_output_aliases=aliases,
    )(*args)


def kernel(inp, plane0, plane1, plane2):
    planes = (plane0, plane1, plane2)
    P = inp.shape[0]
    C = plane0.shape[0]
    D = 4 * C
    tables = [_pack_plane(p) for p in planes]
    cT = inp.T
    w16, idx3 = _prep(cT, [p.shape for p in planes])

    E = np.zeros((16, 3 * D), np.float32)
    for ci in range(3):
        for c in range(4):
            E[4 * ci + c, ci * D + c * C:ci * D + (c + 1) * C] = 1.0
    S = np.zeros((D, C), np.float32)
    for c in range(4):
        S[c * C:(c + 1) * C, :] += np.eye(C, dtype=np.float32)
    E = jnp.asarray(E, jnp.bfloat16)
    S = jnp.asarray(S, jnp.bfloat16)

    pc = P // _NCHUNK
    out = None
    for k in range(_NCHUNK):
        g = _sc_gather3(tables, idx3, k * pc, pc)
        out = _combine(w16, g.reshape(3, pc, D), E, S, k * pc, out)
    return out


# speedup vs baseline: 1.2840x; 1.1926x over previous
import functools

import jax
import jax.numpy as jnp
import numpy as np
from jax.experimental import pallas as pl
from jax.experimental.pallas import tpu as pltpu
from jax.experimental.pallas import tpu_sc as plsc

_COMBS = ((0, 3), (1, 3), (2, 3))
_WINDOW = 256
_BLK = 2048
_BLKP = 16384
_NCHUNK = 8


def _pack_plane(plane):
    t = jnp.transpose(plane, (1, 2, 0))
    tx = jnp.concatenate([t[:, 1:], t[:, -1:]], axis=1)
    ty = jnp.concatenate([t[1:], t[-1:]], axis=0)
    txy = jnp.concatenate([tx[1:], tx[-1:]], axis=0)
    H, W, C = t.shape
    return jnp.concatenate([t, tx, ty, txy], axis=-1).reshape(H * W, 4 * C)


def _prep(cT, plane_shapes):
    P = cT.shape[1]

    def body(c_ref, w_ref, i_ref):
        w_rows = []
        i_rows = []
        for ci, comb in enumerate(_COMBS):
            _, H, W = plane_shapes[ci]
            cx = c_ref[comb[0]:comb[0] + 1, :]
            cy = c_ref[comb[1]:comb[1] + 1, :]
            x = jnp.clip((cx + 1.0) * (0.5 * (W - 1)), 0.0, W - 1)
            y = jnp.clip((cy + 1.0) * (0.5 * (H - 1)), 0.0, H - 1)
            x0 = jnp.floor(x)
            y0 = jnp.floor(y)
            i_rows.append(y0.astype(jnp.int32) * W + x0.astype(jnp.int32))
            wx = x - x0
            wy = y - y0
            w_rows += [(1.0 - wx) * (1.0 - wy), wx * (1.0 - wy),
                       (1.0 - wx) * wy, wx * wy]
        w_rows.append(jnp.zeros((4, w_rows[0].shape[1]), jnp.float32))
        w_ref[...] = jnp.concatenate(w_rows, axis=0).astype(jnp.bfloat16)
        i_ref[...] = jnp.concatenate(i_rows, axis=0)

    return pl.pallas_call(
        body,
        grid=(P // _BLKP,),
        in_specs=[pl.BlockSpec((cT.shape[0], _BLKP), lambda i: (0, i))],
        out_specs=[
            pl.BlockSpec((16, _BLKP), lambda i: (0, i)),
            pl.BlockSpec((3, _BLKP), lambda i: (0, i)),
        ],
        out_shape=[
            jax.ShapeDtypeStruct((16, P), jnp.bfloat16),
            jax.ShapeDtypeStruct((3, P), jnp.int32),
        ],
    )(cT)


def _sc_gather3(tables, idx3, pbase, pcount):
    d = tables[0].shape[1]
    mesh = plsc.VectorSubcoreMesh(core_axis_name="c", subcore_axis_name="s")
    ibase = pbase // _WINDOW

    @functools.partial(
        pl.kernel,
        out_type=jax.ShapeDtypeStruct((3 * pcount, d), tables[0].dtype),
        mesh=mesh,
    )
    def gather_kernel(t0_hbm, t1_hbm, t2_hbm, idx_hbm, out_hbm):
        for ci, t_hbm in enumerate((t0_hbm, t1_hbm, t2_hbm)):
            def body(i_vmem, o_vmem, t_hbm=t_hbm):
                pltpu.sync_copy(t_hbm.at[i_vmem.at[0]], o_vmem)

            obase = ci * (pcount // _WINDOW)
            pltpu.emit_pipeline(
                body,
                grid=(pcount // _WINDOW,),
                in_specs=[pl.BlockSpec((1, _WINDOW),
                                       lambda i, ci=ci: (ci, ibase + i))],
                out_specs=[pl.BlockSpec((_WINDOW, d),
                                        lambda i, obase=obase: (obase + i, 0))],
                core_axis_name=("c", "s"),
                dimension_semantics=(pltpu.PARALLEL,),
            )(idx_hbm, out_hbm)

    return gather_kernel(*tables, idx3)


def _combine(w16, g3, E, S, pbase, prev):
    P = w16.shape[1]
    pcount = g3.shape[1]
    D = g3.shape[2]
    C = S.shape[1]
    wbase = pbase // _BLK

    def body(*refs):
        (w_ref, g_ref, E_ref, S_ref, o_ref) = refs[-5:]
        wt = jnp.transpose(w_ref[...], (1, 0))
        wpat = jax.lax.dot_general(
            wt, E_ref[...], (((1,), (0,)), ((), ())),
            preferred_element_type=jnp.float32)
        acc = g_ref[0] * wpat[:, 0:D]
        acc = acc + g_ref[1] * wpat[:, D:2 * D]
        acc = acc + g_ref[2] * wpat[:, 2 * D:3 * D]
        o_ref[...] = jax.lax.dot_general(
            S_ref[...], acc.astype(jnp.bfloat16), (((0,), (1,)), ((), ())),
            preferred_element_type=jnp.float32)

    in_specs = [
        pl.BlockSpec((16, _BLK), lambda i: (0, wbase + i)),
        pl.BlockSpec((3, _BLK, D), lambda i: (0, i, 0)),
        pl.BlockSpec(E.shape, lambda i: (0, 0)),
        pl.BlockSpec(S.shape, lambda i: (0, 0)),
    ]
    args = (w16, g3, E, S)
    aliases = {}
    if prev is not None:
        in_specs = [pl.BlockSpec(memory_space=pl.ANY)] + in_specs
        args = (prev,) + args
        aliases = {0: 0}
    return pl.pallas_call(
        body,
        grid=(pcount // _BLK,),
        in_specs=in_specs,
        out_specs=pl.BlockSpec((C, _BLK), lambda i: (0, wbase + i)),
        out_shape=jax.ShapeDtypeStruct((C, P), jnp.float32),
        input_output_aliases=aliases,
    )(*args)


def kernel(inp, plane0, plane1, plane2):
    planes = (plane0, plane1, plane2)
    P = inp.shape[0]
    C = plane0.shape[0]
    D = 4 * C
    tables = [_pack_plane(p) for p in planes]
    cT = inp.T
    w16, idx3 = _prep(cT, [p.shape for p in planes])

    E = np.zeros((16, 3 * D), np.float32)
    for ci in range(3):
        for c in range(4):
            E[4 * ci + c, ci * D + c * C:ci * D + (c + 1) * C] = 1.0
    S = np.zeros((D, C), np.float32)
    for c in range(4):
        S[c * C:(c + 1) * C, :] += np.eye(C, dtype=np.float32)
    E = jnp.asarray(E, jnp.bfloat16)
    S = jnp.asarray(S, jnp.bfloat16)

    pc = P // _NCHUNK
    out = None
    for k in range(_NCHUNK):
        g = _sc_gather3(tables, idx3, k * pc, pc)
        out = _combine(w16, g.reshape(3, pc, D), E, S, k * pc, out)
    return out.T
